# baseline (speedup 1.0000x reference)
import functools

import jax
import jax.numpy as jnp
from jax import lax
from jax.experimental import pallas as pl
from jax.experimental.pallas import tpu as pltpu
from jax.experimental.pallas import tpu_sc as plsc

_BATCH = 4096
_HIST = 200
_D = 64
_NW = 32
_BPW = _BATCH // _NW


def _make_sc_gather():
    mesh = plsc.VectorSubcoreMesh(core_axis_name="c", subcore_axis_name="s")

    @functools.partial(
        pl.kernel,
        mesh=mesh,
        compiler_params=pltpu.CompilerParams(use_tc_tiling_on_sc=False),
        out_type=jax.ShapeDtypeStruct((_BATCH, _HIST, _D), jnp.float32),
        scratch_types=[
            pltpu.VMEM((_HIST,), jnp.int32),
            pltpu.VMEM((_HIST,), jnp.int32),
            pltpu.VMEM((_HIST, _D), jnp.float32),
            pltpu.VMEM((_HIST, _D), jnp.float32),
            pltpu.SemaphoreType.DMA,
            pltpu.SemaphoreType.DMA,
            pltpu.SemaphoreType.DMA,
            pltpu.SemaphoreType.DMA,
        ],
    )
    def k(idx_hbm, table_hbm, out_hbm, idx0, idx1, rows0, rows1,
          gs0, gs1, ws0, ws1):
        wid = lax.axis_index("s") * 2 + lax.axis_index("c")
        base = wid * _BPW
        idxb = (idx0, idx1)
        rowsb = (rows0, rows1)
        gs = (gs0, gs1)
        ws = (ws0, ws1)

        def issue_gather(g, b):
            pltpu.sync_copy(idx_hbm.at[base + g], idxb[b])
            pltpu.async_copy(table_hbm.at[idxb[b]], rowsb[b], gs[b])

        def wait_gather(b):
            pltpu.make_async_copy(table_hbm.at[idxb[b]], rowsb[b], gs[b]).wait()

        def issue_write(g, b):
            pltpu.async_copy(rowsb[b], out_hbm.at[base + g], ws[b])

        def wait_write(g, b):
            pltpu.make_async_copy(rowsb[b], out_hbm.at[base + g], ws[b]).wait()

        issue_gather(0, 0)

        def body(i, carry):
            for b in range(2):
                g = i * 2 + b
                nb = (b + 1) % 2
                wait_gather(b)
                issue_write(g, b)

                @pl.when(g >= 1)
                def _():
                    wait_write(g - 1, nb)

                @pl.when(g + 1 < _BPW)
                def _():
                    issue_gather(g + 1, nb)
            return carry

        lax.fori_loop(0, _BPW // 2, body, 0)
        wait_write(_BPW - 1, (_BPW - 1) % 2)

    return k


_sc_gather = _make_sc_gather()


def kernel(timestamps, table):
    return _sc_gather(timestamps, table)
